# baseline (speedup 1.0000x reference)
import jax
import jax.numpy as jnp
from jax import lax
from jax.experimental import pallas as pl
from jax.experimental.pallas import tpu as pltpu
from jax.experimental.pallas import tpu_sc as plsc

N_TOK = 16384
D = 128
C0, C1, C2, C3 = 10000, 60000, 190000, 1000000
WB1 = C0
WB2 = WB1 + (C1 - C0) * 32 // 128
WB3 = WB2 + (C2 - C1) * 8 // 128
N_WIN = WB3 + ((C3 - C2) * 2 + 127) // 128
PAD = N_WIN * 128 - (C0 * 128 + (C1 - C0) * 32 + (C2 - C1) * 8 + (C3 - C2) * 2)

NW = 32
B_W = N_TOK // NW
G = 4
B_G = B_W // G

SIZES = (C0 * 128, (C1 - C0) * 32, (C2 - C1) * 8, (C3 - C2) * 2)
BASES = (0, SIZES[0], SIZES[0] + SIZES[1], SIZES[0] + SIZES[1] + SIZES[2])
FLAT_LEN = N_WIN * 128
SHARES = tuple(((s // 16 + NW - 1) // NW) * 16 for s in SIZES)


def _concat_body(t0, t1, t2, t3, flat, sem):
    wid = lax.axis_index("s") * 2 + lax.axis_index("c")
    copies = []
    for src, size, base, share in zip((t0, t1, t2, t3), SIZES, BASES, SHARES):
        off = (wid * (size // 16) // NW) * 16
        off = jnp.minimum(off, size - share)
        copies.append(pltpu.make_async_copy(
            src.at[pl.ds(off, share)], flat.at[pl.ds(base + off, share)], sem))
    for c in copies:
        c.start()
    for c in copies:
        c.wait()


@jax.jit
def _sc_concat(t0, t1, t2, t3):
    return pl.kernel(
        _concat_body,
        out_type=jax.ShapeDtypeStruct((FLAT_LEN,), jnp.float32),
        mesh=plsc.VectorSubcoreMesh(core_axis_name="c", subcore_axis_name="s"),
        compiler_params=pltpu.CompilerParams(use_tc_tiling_on_sc=False),
        scratch_types=[pltpu.SemaphoreType.DMA],
    )(t0, t1, t2, t3)


def _sc_body(inp, flat, out, idx_v, win_v, buf, sem):
    wid = lax.axis_index("s") * 2 + lax.axis_index("c")
    base = wid * B_W
    pltpu.sync_copy(inp.at[pl.ds(base, B_W)], idx_v)
    for i in range(B_W // 16):
        v = idx_v[pl.ds(i * 16, 16)]
        w = jnp.where(
            v < C0, v,
            jnp.where(
                v < C1, WB1 + lax.shift_right_logical(v - C0, 2),
                jnp.where(
                    v < C2, WB2 + lax.shift_right_logical(v - C1, 4),
                    WB3 + lax.shift_right_logical(v - C2, 6))))
        win_v[i // 8, pl.ds((i % 8) * 16, 16)] = w
    copies = [
        pltpu.make_async_copy(flat.at[win_v.at[j]],
                              buf.at[pl.ds(j * B_G, B_G)], sem)
        for j in range(G)
    ]
    for c in copies:
        c.start()
    for c in copies:
        c.wait()
    pltpu.sync_copy(buf, out.at[pl.ds(base, B_W)])


@jax.jit
def _sc_gather(inp, flat):
    return pl.kernel(
        _sc_body,
        out_type=jax.ShapeDtypeStruct((N_TOK, 128), jnp.float32),
        mesh=plsc.VectorSubcoreMesh(core_axis_name="c", subcore_axis_name="s"),
        compiler_params=pltpu.CompilerParams(use_tc_tiling_on_sc=False),
        scratch_types=[
            pltpu.VMEM((B_W,), jnp.int32),
            pltpu.VMEM((G, B_G), jnp.int32),
            pltpu.VMEM((B_W, 128), jnp.float32),
            pltpu.SemaphoreType.DMA,
        ],
    )(inp, flat)


B_M = 1024


def _mm_body(inp, gw, hw, w1d, w2d, w3d, out):
    v = inp[...]
    g = gw[...]
    lane = lax.broadcasted_iota(jnp.int32, (B_M, 128), 1)
    hm = jnp.where(v < C0, g, 0.0)
    t1m = jnp.where((v >= C0) & (v < C1) & ((lane >> 5) == ((v - C0) & 3)),
                    g, 0.0)
    t2m = jnp.where((v >= C1) & (v < C2) & ((lane >> 3) == ((v - C1) & 15)),
                    g, 0.0)
    t3m = jnp.where((v >= C2) & ((lane >> 1) == ((v - C2) & 63)), g, 0.0)
    acc = jnp.dot(hm, hw[...], preferred_element_type=jnp.float32)
    acc += jnp.dot(t1m, w1d[...], preferred_element_type=jnp.float32)
    acc += jnp.dot(t2m, w2d[...], preferred_element_type=jnp.float32)
    acc += jnp.dot(t3m, w3d[...], preferred_element_type=jnp.float32)
    out[...] = acc


@jax.jit
def _tc_project(inp2, gw, hw, w1d, w2d, w3d):
    nb = N_TOK // B_M
    blk = lambda r: pl.BlockSpec((B_M, r), lambda b: (b, 0))
    full = lambda a, b: pl.BlockSpec((a, b), lambda _: (0, 0))
    return pl.pallas_call(
        _mm_body,
        grid=(nb,),
        in_specs=[blk(1), blk(128),
                  full(128, 128), full(128, 128), full(128, 128),
                  full(128, 128)],
        out_specs=blk(128),
        out_shape=jax.ShapeDtypeStruct((N_TOK, D), jnp.float32),
    )(inp2, gw, hw, w1d, w2d, w3d)


def kernel(input, head_emb, head_W, emb1, W1, emb2, W2, emb3, W3):
    flat = _sc_concat(head_emb.reshape(-1), emb1.reshape(-1),
                      emb2.reshape(-1), emb3.reshape(-1))
    gw = _sc_gather(input, flat.reshape(N_WIN, 128))
    w1d = jnp.tile(W1, (4, 1))
    w2d = jnp.tile(W2, (16, 1))
    w3d = jnp.tile(W3, (64, 1))
    return _tc_project(input.reshape(N_TOK, 1), gw,
                       head_W, w1d, w2d, w3d)
